# baseline (speedup 1.0000x reference)
import functools

import jax
import jax.numpy as jnp
from jax import lax
from jax.experimental import pallas as pl
from jax.experimental.pallas import tpu as pltpu
from jax.experimental.pallas import tpu_sc as plsc

_BF = jnp.bfloat16
_F32 = jnp.float32

_GW = 128
_SW = 128


def _sc_gather(table, idx2):
    num_idx = idx2.shape[1]
    d = table.shape[1]

    @functools.partial(
        pl.kernel,
        out_type=jax.ShapeDtypeStruct((num_idx, d), table.dtype),
        mesh=plsc.VectorSubcoreMesh(core_axis_name="c", subcore_axis_name="s"),
    )
    def gat(tbl_hbm, i_hbm, o_hbm):
        def body(i_vmem, o_vmem):
            pltpu.sync_copy(tbl_hbm.at[i_vmem.at[0]], o_vmem)

        pltpu.emit_pipeline(
            body,
            grid=(num_idx // _GW,),
            in_specs=[pl.BlockSpec((1, _GW), lambda i: (0, i))],
            out_specs=[pl.BlockSpec((_GW, d), lambda i: (i, 0))],
            core_axis_name=("c", "s"),
            dimension_semantics=(pltpu.PARALLEL,),
        )(i_hbm, o_hbm)

    return gat(table, idx2)


def _take16(v, idx):
    dn = lax.GatherDimensionNumbers(
        offset_dims=(), collapsed_slice_dims=(0,), start_index_map=(0,))
    return lax.gather(v, idx[:, None], dn, (1,),
                      mode=lax.GatherScatterMode.PROMISE_IN_BOUNDS)


_RL = 320
_CAP = 5696
_GB = 64


def _take16(v, idx):
    dn = lax.GatherDimensionNumbers(
        offset_dims=(), collapsed_slice_dims=(0,), start_index_map=(0,))
    return lax.gather(v, idx[:, None], dn, (1,),
                      mode=lax.GatherScatterMode.PROMISE_IN_BOUNDS)


_RL = 320
_CAP = 5696
_GB = 64
_CH = 1280


def _sc_build_lists(dst2):
    e_total = dst2.shape[1]
    nvec = 16
    nch = e_total // _CH

    @functools.partial(
        pl.kernel,
        out_type=(
            jax.ShapeDtypeStruct((32 * _CAP,), jnp.int32),
            jax.ShapeDtypeStruct((32 * _CAP,), jnp.int32),
            jax.ShapeDtypeStruct((32 * nvec,), jnp.int32),
        ),
        mesh=plsc.VectorSubcoreMesh(core_axis_name="c", subcore_axis_name="s"),
        compiler_params=pltpu.CompilerParams(needs_layout_passes=False),
        scratch_types=[
            pltpu.VMEM((_CAP,), jnp.int32),
            pltpu.VMEM((_CAP,), jnp.int32),
            pltpu.VMEM((2, 1, _CH), jnp.int32),
            pltpu.VMEM((nvec,), jnp.int32),
            pltpu.SemaphoreType.DMA,
            pltpu.SemaphoreType.DMA,
        ],
    )
    def build(dst_hbm, o_ids, o_locs, o_cnt, ids, locs, dbuf, offc, sem0,
              sem1):
        c = lax.axis_index("c")
        s = lax.axis_index("s")
        w = s * 2 + c
        base = w * _RL
        iota = lax.iota(jnp.int32, nvec)
        zi = jnp.zeros((nvec,), jnp.int32)

        @pl.loop(0, _CAP // nvec)
        def _(i):
            ids[pl.ds(i * nvec, nvec)] = zi

        offc[...] = zi
        sems = (sem0, sem1)

        def copy_a(k, b):
            return pltpu.make_async_copy(
                dst_hbm.at[:, pl.ds(k * _CH, _CH)], dbuf.at[b], sems[b])

        def proc_a(k, b):
            dr = dbuf.at[b].at[0]

            @pl.loop(0, _CH // nvec)
            def _(j):
                dv = dr[pl.ds(j * nvec, nvec)]
                eid = (k * _CH + j * nvec) + iota
                loc = dv - base
                msk = plsc.bitcast(loc, jnp.uint32) < jnp.uint32(_RL)
                mi = jnp.where(msk, 1, 0)
                cs = plsc.cumsum(mi)
                off = offc[...]
                pos = off + cs - mi
                plsc.store_scatter(ids, [pos], eid, mask=msk)
                plsc.store_scatter(locs, [pos], loc, mask=msk)
                offc[...] = off + plsc.all_reduce_population_count(msk)

        copy_a(0, 0).start()

        @pl.loop(0, (nch + 1) // 2)
        def _(u):
            k0 = 2 * u

            @pl.when(k0 + 1 < nch)
            def _():
                copy_a(k0 + 1, 1).start()

            copy_a(k0, 0).wait()
            proc_a(k0, 0)

            @pl.when(k0 + 2 < nch)
            def _():
                copy_a(k0 + 2, 0).start()

            @pl.when(k0 + 1 < nch)
            def _():
                copy_a(k0 + 1, 1).wait()
                proc_a(k0 + 1, 1)

        pltpu.sync_copy(ids, o_ids.at[pl.ds(w * _CAP, _CAP)])
        pltpu.sync_copy(locs, o_locs.at[pl.ds(w * _CAP, _CAP)])
        pltpu.sync_copy(offc, o_cnt.at[pl.ds(w * nvec, nvec)])

    return build(dst2)


def _sc_scatter_add(weighted, ids_all, locs_all, cnt_all):
    e_total, d = weighted.shape
    nvec = 16
    ngrp = _CAP // _GB

    @functools.partial(
        pl.kernel,
        out_type=jax.ShapeDtypeStruct((32 * _RL, d), _F32),
        mesh=plsc.VectorSubcoreMesh(core_axis_name="c", subcore_axis_name="s"),
        compiler_params=pltpu.CompilerParams(needs_layout_passes=False),
        scratch_types=[
            pltpu.VMEM((_RL, d), _F32),
            pltpu.VMEM((_CAP,), jnp.int32),
            pltpu.VMEM((_CAP,), jnp.int32),
            pltpu.VMEM((2, _GB, d), _F32),
            pltpu.VMEM((nvec,), jnp.int32),
            pltpu.SemaphoreType.DMA,
            pltpu.SemaphoreType.DMA,
        ],
    )
    def scat(w_hbm, ids_hbm, locs_hbm, cnt_hbm, o_hbm, acc, ids, locs, rbuf,
             offc, sem0, sem1):
        c = lax.axis_index("c")
        s = lax.axis_index("s")
        w = s * 2 + c
        base = w * _RL
        iota = lax.iota(jnp.int32, nvec)
        zf = jnp.zeros((nvec,), _F32)
        sems = (sem0, sem1)

        pltpu.sync_copy(ids_hbm.at[pl.ds(w * _CAP, _CAP)], ids)
        pltpu.sync_copy(locs_hbm.at[pl.ds(w * _CAP, _CAP)], locs)
        pltpu.sync_copy(cnt_hbm.at[pl.ds(w * nvec, nvec)], offc)

        @pl.loop(0, _RL)
        def _(r):
            rv = iota * 0 + r

            @pl.loop(0, d // nvec)
            def _(g):
                plsc.store_scatter(acc, [rv, g * nvec + iota], zf)

        cnt = jnp.max(offc[...])
        cntv = iota * 0 + cnt

        def copy_b(g, b):
            return pltpu.make_async_copy(
                w_hbm.at[ids.at[pl.ds(g * _GB, _GB)]], rbuf.at[b], sems[b])

        def proc_b(g, b):
            rb = rbuf.at[b]
            for h in range(_GB // nvec):
                lv = locs[pl.ds(g * _GB + h * nvec, nvec)]
                for e in range(nvec):
                    q = g * _GB + h * nvec + e
                    racc = _take16(lv, iota * 0 + e)
                    valid = (iota * 0 + q) < cntv
                    rsrc = iota * 0 + (h * nvec + e)
                    for ch in range(d // nvec):
                        colv = ch * nvec + iota
                        x = plsc.load_gather(rb, [rsrc, colv])
                        plsc.addupdate_scatter(acc, [racc, colv], x,
                                               mask=valid)

        @pl.when(0 < cnt)
        def _():
            copy_b(0, 0).start()

        @pl.loop(0, ngrp // 2)
        def _(u):
            g0 = 2 * u

            @pl.when((g0 + 1) * _GB < cnt)
            def _():
                copy_b(g0 + 1, 1).start()

            @pl.when(g0 * _GB < cnt)
            def _():
                copy_b(g0, 0).wait()
                proc_b(g0, 0)

            @pl.when((g0 + 2) * _GB < cnt)
            def _():
                copy_b(g0 + 2, 0).start()

            @pl.when((g0 + 1) * _GB < cnt)
            def _():
                copy_b(g0 + 1, 1).wait()
                proc_b(g0 + 1, 1)

        pltpu.sync_copy(acc, o_hbm.at[pl.ds(base, _RL)])

    return scat(weighted, ids_all, locs_all, cnt_all)


def _edge_body(he_ref, gs_ref, gd_ref, norm_ref, we_ref, ws_ref, wd_ref,
               b1a_ref, w1b_ref, b1b_ref, out_he_ref, out_w_ref):
    he = he_ref[...]
    x = jnp.dot(gs_ref[0].astype(_BF), ws_ref[...],
                preferred_element_type=_F32)
    x += jnp.dot(gd_ref[0].astype(_BF), wd_ref[...],
                 preferred_element_type=_F32)
    x += jnp.dot(he.astype(_BF), we_ref[...], preferred_element_type=_F32)
    x += b1a_ref[...]
    t = jnp.tanh(x)
    h = jnp.dot(t.astype(_BF), w1b_ref[...], preferred_element_type=_F32)
    he_new = he + (h + b1b_ref[...])
    out_he_ref[...] = he_new
    out_w_ref[...] = he_new * norm_ref[...]


def _tc_edge(he, g3, norm2, we, ws, wd, b1a2, w1b, b1b2):
    e_total, d = he.shape
    be = 640
    row = pl.BlockSpec((be, d), lambda i: (i, 0))
    wspec = pl.BlockSpec((d, d), lambda i: (0, 0))
    bspec = pl.BlockSpec((1, d), lambda i: (0, 0))
    return pl.pallas_call(
        _edge_body,
        grid=(e_total // be,),
        in_specs=[
            row,
            pl.BlockSpec((1, be, d), lambda i: (0, i, 0)),
            pl.BlockSpec((1, be, d), lambda i: (1, i, 0)),
            pl.BlockSpec((be, 1), lambda i: (i, 0)),
            wspec, wspec, wspec, bspec, wspec, bspec,
        ],
        out_specs=[row, row],
        out_shape=[jax.ShapeDtypeStruct((e_total, d), _F32)] * 2,
    )(he, g3, g3, norm2, we, ws, wd, b1a2, w1b, b1b2)


def _node_body(hn_ref, nf_ref, wh_ref, wf_ref, b2a_ref, w2b_ref, b2b_ref,
               out_ref):
    hn = hn_ref[...]
    x = jnp.dot(hn.astype(_BF), wh_ref[...], preferred_element_type=_F32)
    x += jnp.dot(nf_ref[...].astype(_BF), wf_ref[...], preferred_element_type=_F32)
    x += b2a_ref[...]
    t = jnp.tanh(x)
    h = jnp.dot(t.astype(_BF), w2b_ref[...], preferred_element_type=_F32)
    out_ref[...] = hn + (h + b2b_ref[...])


def _tc_node(hn, node_ftr, wh, wf, b2a2, w2b, b2b2):
    n, d = hn.shape
    bn = 1000
    row = pl.BlockSpec((bn, d), lambda i: (i, 0))
    wspec = pl.BlockSpec((d, d), lambda i: (0, 0))
    bspec = pl.BlockSpec((1, d), lambda i: (0, 0))
    return pl.pallas_call(
        _node_body,
        grid=(n // bn,),
        in_specs=[row, row, wspec, wspec, bspec, wspec, bspec],
        out_specs=row,
        out_shape=jax.ShapeDtypeStruct((n, d), _F32),
    )(hn, node_ftr, wh, wf, b2a2, w2b, b2b2)


def kernel(hn, he, edge_index, norm, W1a, b1a, W1b, b1b, W2a, b2a, W2b, b2b):
    n, d = hn.shape
    e_total = he.shape[0]
    src = edge_index[0].astype(jnp.int32)
    dst = edge_index[1].astype(jnp.int32)

    ids_all, locs_all, cnt_all = _sc_build_lists(dst.reshape(1, e_total))
    idx2 = jnp.concatenate([src, dst]).reshape(1, 2 * e_total)
    gathered = _sc_gather(hn, idx2)
    g3 = gathered.reshape(2, e_total, d)

    we = W1a[:d].astype(_BF)
    ws = W1a[d:2 * d].astype(_BF)
    wd = W1a[2 * d:].astype(_BF)
    he_new, weighted = _tc_edge(
        he, g3, norm.reshape(e_total, 1), we, ws, wd,
        b1a.reshape(1, d), W1b.astype(_BF), b1b.reshape(1, d))

    padded = _sc_scatter_add(weighted, ids_all, locs_all, cnt_all)
    node_ftr = padded[:n]

    wh = W2a[:d].astype(_BF)
    wf = W2a[d:].astype(_BF)
    hn_new = _tc_node(hn, node_ftr, wh, wf, b2a.reshape(1, d),
                      W2b.astype(_BF), b2b.reshape(1, d))
    return he_new, hn_new


# speedup vs baseline: 1.0447x; 1.0447x over previous
import functools

import jax
import jax.numpy as jnp
from jax import lax
from jax.experimental import pallas as pl
from jax.experimental.pallas import tpu as pltpu
from jax.experimental.pallas import tpu_sc as plsc

_BF = jnp.bfloat16
_F32 = jnp.float32

_GW = 128
_SW = 128


def _sc_gather(table, idx2):
    num_idx = idx2.shape[1]
    d = table.shape[1]

    @functools.partial(
        pl.kernel,
        out_type=jax.ShapeDtypeStruct((num_idx, d), table.dtype),
        mesh=plsc.VectorSubcoreMesh(core_axis_name="c", subcore_axis_name="s"),
    )
    def gat(tbl_hbm, i_hbm, o_hbm):
        def body(i_vmem, o_vmem):
            pltpu.sync_copy(tbl_hbm.at[i_vmem.at[0]], o_vmem)

        pltpu.emit_pipeline(
            body,
            grid=(num_idx // _GW,),
            in_specs=[pl.BlockSpec((1, _GW), lambda i: (0, i))],
            out_specs=[pl.BlockSpec((_GW, d), lambda i: (i, 0))],
            core_axis_name=("c", "s"),
            dimension_semantics=(pltpu.PARALLEL,),
        )(i_hbm, o_hbm)

    return gat(table, idx2)


def _take16(v, idx):
    dn = lax.GatherDimensionNumbers(
        offset_dims=(), collapsed_slice_dims=(0,), start_index_map=(0,))
    return lax.gather(v, idx[:, None], dn, (1,),
                      mode=lax.GatherScatterMode.PROMISE_IN_BOUNDS)


_RL = 320
_CAP = 5696
_GB = 32


def _take16(v, idx):
    dn = lax.GatherDimensionNumbers(
        offset_dims=(), collapsed_slice_dims=(0,), start_index_map=(0,))
    return lax.gather(v, idx[:, None], dn, (1,),
                      mode=lax.GatherScatterMode.PROMISE_IN_BOUNDS)


_RL = 320
_CAP = 5696
_GB = 32
_CH = 1280


def _sc_build_lists(dst2):
    e_total = dst2.shape[1]
    nvec = 16
    nch = e_total // _CH

    @functools.partial(
        pl.kernel,
        out_type=(
            jax.ShapeDtypeStruct((32 * _CAP,), jnp.int32),
            jax.ShapeDtypeStruct((32 * _CAP,), jnp.int32),
            jax.ShapeDtypeStruct((32 * nvec,), jnp.int32),
        ),
        mesh=plsc.VectorSubcoreMesh(core_axis_name="c", subcore_axis_name="s"),
        compiler_params=pltpu.CompilerParams(needs_layout_passes=False),
        scratch_types=[
            pltpu.VMEM((_CAP,), jnp.int32),
            pltpu.VMEM((_CAP,), jnp.int32),
            pltpu.VMEM((2, 1, _CH), jnp.int32),
            pltpu.VMEM((nvec,), jnp.int32),
            pltpu.SemaphoreType.DMA,
            pltpu.SemaphoreType.DMA,
        ],
    )
    def build(dst_hbm, o_ids, o_locs, o_cnt, ids, locs, dbuf, offc, sem0,
              sem1):
        c = lax.axis_index("c")
        s = lax.axis_index("s")
        w = s * 2 + c
        base = w * _RL
        iota = lax.iota(jnp.int32, nvec)
        zi = jnp.zeros((nvec,), jnp.int32)

        @pl.loop(0, _CAP // nvec)
        def _(i):
            ids[pl.ds(i * nvec, nvec)] = zi

        offc[...] = zi
        sems = (sem0, sem1)

        def copy_a(k, b):
            return pltpu.make_async_copy(
                dst_hbm.at[:, pl.ds(k * _CH, _CH)], dbuf.at[b], sems[b])

        def proc_a(k, b):
            dr = dbuf.at[b].at[0]

            @pl.loop(0, _CH // nvec)
            def _(j):
                dv = dr[pl.ds(j * nvec, nvec)]
                eid = (k * _CH + j * nvec) + iota
                loc = dv - base
                msk = plsc.bitcast(loc, jnp.uint32) < jnp.uint32(_RL)
                mi = jnp.where(msk, 1, 0)
                cs = plsc.cumsum(mi)
                off = offc[...]
                pos = off + cs - mi
                plsc.store_scatter(ids, [pos], eid, mask=msk)
                plsc.store_scatter(locs, [pos], loc, mask=msk)
                offc[...] = off + plsc.all_reduce_population_count(msk)

        copy_a(0, 0).start()

        @pl.loop(0, (nch + 1) // 2)
        def _(u):
            k0 = 2 * u

            @pl.when(k0 + 1 < nch)
            def _():
                copy_a(k0 + 1, 1).start()

            copy_a(k0, 0).wait()
            proc_a(k0, 0)

            @pl.when(k0 + 2 < nch)
            def _():
                copy_a(k0 + 2, 0).start()

            @pl.when(k0 + 1 < nch)
            def _():
                copy_a(k0 + 1, 1).wait()
                proc_a(k0 + 1, 1)

        pltpu.sync_copy(ids, o_ids.at[pl.ds(w * _CAP, _CAP)])
        pltpu.sync_copy(locs, o_locs.at[pl.ds(w * _CAP, _CAP)])
        pltpu.sync_copy(offc, o_cnt.at[pl.ds(w * nvec, nvec)])

    return build(dst2)


def _sc_scatter_add(weighted, ids_all, locs_all, cnt_all):
    e_total, d = weighted.shape
    nvec = 16
    ngrp = _CAP // _GB

    @functools.partial(
        pl.kernel,
        out_type=jax.ShapeDtypeStruct((32 * _RL, d), _F32),
        mesh=plsc.VectorSubcoreMesh(core_axis_name="c", subcore_axis_name="s"),
        compiler_params=pltpu.CompilerParams(needs_layout_passes=False),
        scratch_types=[
            pltpu.VMEM((_RL, d), _F32),
            pltpu.VMEM((_CAP,), jnp.int32),
            pltpu.VMEM((_CAP,), jnp.int32),
            pltpu.VMEM((2, _GB, d), _F32),
            pltpu.VMEM((nvec,), jnp.int32),
            pltpu.SemaphoreType.DMA,
            pltpu.SemaphoreType.DMA,
        ],
    )
    def scat(w_hbm, ids_hbm, locs_hbm, cnt_hbm, o_hbm, acc, ids, locs, rbuf,
             offc, sem0, sem1):
        c = lax.axis_index("c")
        s = lax.axis_index("s")
        w = s * 2 + c
        base = w * _RL
        iota = lax.iota(jnp.int32, nvec)
        zf = jnp.zeros((nvec,), _F32)
        sems = (sem0, sem1)

        pltpu.sync_copy(ids_hbm.at[pl.ds(w * _CAP, _CAP)], ids)
        pltpu.sync_copy(locs_hbm.at[pl.ds(w * _CAP, _CAP)], locs)
        pltpu.sync_copy(cnt_hbm.at[pl.ds(w * nvec, nvec)], offc)

        @pl.loop(0, _RL)
        def _(r):
            rv = iota * 0 + r

            @pl.loop(0, d // nvec)
            def _(g):
                plsc.store_scatter(acc, [rv, g * nvec + iota], zf)

        cnt = jnp.max(offc[...])
        cntv = iota * 0 + cnt

        def copy_b(g, b):
            return pltpu.make_async_copy(
                w_hbm.at[ids.at[pl.ds(g * _GB, _GB)]], rbuf.at[b], sems[b])

        def proc_b(g, b):
            rb = rbuf.at[b]
            for h in range(_GB // nvec):
                lv = locs[pl.ds(g * _GB + h * nvec, nvec)]
                for e in range(nvec):
                    q = g * _GB + h * nvec + e
                    racc = _take16(lv, iota * 0 + e)
                    valid = (iota * 0 + q) < cntv
                    rsrc = iota * 0 + (h * nvec + e)
                    for ch in range(d // nvec):
                        colv = ch * nvec + iota
                        x = plsc.load_gather(rb, [rsrc, colv])
                        plsc.addupdate_scatter(acc, [racc, colv], x,
                                               mask=valid)

        @pl.when(0 < cnt)
        def _():
            copy_b(0, 0).start()

        @pl.loop(0, ngrp // 2)
        def _(u):
            g0 = 2 * u

            @pl.when((g0 + 1) * _GB < cnt)
            def _():
                copy_b(g0 + 1, 1).start()

            @pl.when(g0 * _GB < cnt)
            def _():
                copy_b(g0, 0).wait()
                proc_b(g0, 0)

            @pl.when((g0 + 2) * _GB < cnt)
            def _():
                copy_b(g0 + 2, 0).start()

            @pl.when((g0 + 1) * _GB < cnt)
            def _():
                copy_b(g0 + 1, 1).wait()
                proc_b(g0 + 1, 1)

        pltpu.sync_copy(acc, o_hbm.at[pl.ds(base, _RL)])

    return scat(weighted, ids_all, locs_all, cnt_all)


def _edge_body(he_ref, gs_ref, gd_ref, norm_ref, we_ref, ws_ref, wd_ref,
               b1a_ref, w1b_ref, b1b_ref, out_he_ref, out_w_ref):
    he = he_ref[...]
    x = jnp.dot(gs_ref[0].astype(_BF), ws_ref[...],
                preferred_element_type=_F32)
    x += jnp.dot(gd_ref[0].astype(_BF), wd_ref[...],
                 preferred_element_type=_F32)
    x += jnp.dot(he.astype(_BF), we_ref[...], preferred_element_type=_F32)
    x += b1a_ref[...]
    t = jnp.tanh(x)
    h = jnp.dot(t.astype(_BF), w1b_ref[...], preferred_element_type=_F32)
    he_new = he + (h + b1b_ref[...])
    out_he_ref[...] = he_new
    out_w_ref[...] = he_new * norm_ref[...]


def _tc_edge(he, g3, norm2, we, ws, wd, b1a2, w1b, b1b2):
    e_total, d = he.shape
    be = 640
    row = pl.BlockSpec((be, d), lambda i: (i, 0))
    wspec = pl.BlockSpec((d, d), lambda i: (0, 0))
    bspec = pl.BlockSpec((1, d), lambda i: (0, 0))
    return pl.pallas_call(
        _edge_body,
        grid=(e_total // be,),
        in_specs=[
            row,
            pl.BlockSpec((1, be, d), lambda i: (0, i, 0)),
            pl.BlockSpec((1, be, d), lambda i: (1, i, 0)),
            pl.BlockSpec((be, 1), lambda i: (i, 0)),
            wspec, wspec, wspec, bspec, wspec, bspec,
        ],
        out_specs=[row, row],
        out_shape=[jax.ShapeDtypeStruct((e_total, d), _F32)] * 2,
    )(he, g3, g3, norm2, we, ws, wd, b1a2, w1b, b1b2)


def _node_body(hn_ref, nf_ref, wh_ref, wf_ref, b2a_ref, w2b_ref, b2b_ref,
               out_ref):
    hn = hn_ref[...]
    x = jnp.dot(hn.astype(_BF), wh_ref[...], preferred_element_type=_F32)
    x += jnp.dot(nf_ref[...].astype(_BF), wf_ref[...], preferred_element_type=_F32)
    x += b2a_ref[...]
    t = jnp.tanh(x)
    h = jnp.dot(t.astype(_BF), w2b_ref[...], preferred_element_type=_F32)
    out_ref[...] = hn + (h + b2b_ref[...])


def _tc_node(hn, node_ftr, wh, wf, b2a2, w2b, b2b2):
    n, d = hn.shape
    bn = 1000
    row = pl.BlockSpec((bn, d), lambda i: (i, 0))
    wspec = pl.BlockSpec((d, d), lambda i: (0, 0))
    bspec = pl.BlockSpec((1, d), lambda i: (0, 0))
    return pl.pallas_call(
        _node_body,
        grid=(n // bn,),
        in_specs=[row, row, wspec, wspec, bspec, wspec, bspec],
        out_specs=row,
        out_shape=jax.ShapeDtypeStruct((n, d), _F32),
    )(hn, node_ftr, wh, wf, b2a2, w2b, b2b2)


def kernel(hn, he, edge_index, norm, W1a, b1a, W1b, b1b, W2a, b2a, W2b, b2b):
    n, d = hn.shape
    e_total = he.shape[0]
    src = edge_index[0].astype(jnp.int32)
    dst = edge_index[1].astype(jnp.int32)

    ids_all, locs_all, cnt_all = _sc_build_lists(dst.reshape(1, e_total))
    idx2 = jnp.concatenate([src, dst]).reshape(1, 2 * e_total)
    gathered = _sc_gather(hn, idx2)
    g3 = gathered.reshape(2, e_total, d)

    we = W1a[:d].astype(_BF)
    ws = W1a[d:2 * d].astype(_BF)
    wd = W1a[2 * d:].astype(_BF)
    he_new, weighted = _tc_edge(
        he, g3, norm.reshape(e_total, 1), we, ws, wd,
        b1a.reshape(1, d), W1b.astype(_BF), b1b.reshape(1, d))

    padded = _sc_scatter_add(weighted, ids_all, locs_all, cnt_all)
    node_ftr = padded[:n]

    wh = W2a[:d].astype(_BF)
    wf = W2a[d:].astype(_BF)
    hn_new = _tc_node(hn, node_ftr, wh, wf, b2a.reshape(1, d),
                      W2b.astype(_BF), b2b.reshape(1, d))
    return he_new, hn_new
